# baseline (speedup 1.0000x reference)
import functools

import jax
import jax.numpy as jnp
import numpy as np
from jax import lax
from jax.experimental import pallas as pl
from jax.experimental.pallas import tpu as pltpu
from jax.experimental.pallas import tpu_sc as plsc

M = 4096
N = 32768
L = 16
D = 12
DP = 16
IMAX = np.int32(2**31 - 1)


def _searchsorted(ot_v, target):
    pos = jnp.zeros((L,), jnp.int32)
    bit = N
    while bit >= 1:
        nxt = pos + bit
        ok = nxt <= N
        idx = jnp.minimum(nxt, N) - 1
        vals = plsc.load_gather(ot_v, [idx])
        pos = jnp.where(ok & (vals < target), nxt, pos)
        bit //= 2
    return pos


def _body(nc, qpw, vt_hbm, ot_hbm, odom_hbm, out_hbm,
          ot_v, vt_v, a_v, b_v, am_v, bm_v, w0_v, w1_v,
          rows0_v, rows1_v, out_v, sem0, sem1):
    wid = lax.axis_index("s") * nc + lax.axis_index("c")
    base = wid * qpw
    with jax.named_scope("stage_table"):
        pltpu.sync_copy(ot_hbm, ot_v)
        pltpu.sync_copy(vt_hbm.at[pl.ds(base, qpw)], vt_v)

    _scope = jax.named_scope("search")
    _scope.__enter__()
    for k in range(qpw // L):
        vt16 = vt_v[pl.ds(k * L, L)]
        pL = _searchsorted(ot_v, vt16)
        wL = plsc.load_gather(ot_v, [jnp.maximum(pL, 1) - 1])
        wR = plsc.load_gather(ot_v, [jnp.minimum(pL, N - 1)])
        dL = vt16 - wL
        dR = jnp.where(pL < N, wR - vt16, IMAX)
        takeL = dL <= dR
        first_wL = _searchsorted(ot_v, wL)
        ref = jnp.where(takeL, first_wL, pL)
        d = jnp.where(takeL, dL, -dR)
        step = (d > 0).astype(jnp.int32) - (d < 0).astype(jnp.int32)
        q = jnp.clip(ref + step, 0, M - 1)
        a = jnp.minimum(ref, q)
        b = jnp.maximum(ref, q)
        x0 = plsc.load_gather(ot_v, [a])
        x1 = plsc.load_gather(ot_v, [b])
        eq = x0 == x1
        x0f = x0.astype(jnp.float32)
        x1f = x1.astype(jnp.float32)
        vtf = vt16.astype(jnp.float32)
        denom = jnp.where(eq, jnp.float32(1.0), x1f - x0f)
        w0 = 1.0 - (vtf - x0f) / denom
        w1 = 1.0 - w0
        w0 = jnp.where(eq, jnp.float32(1.0), w0)
        w1 = jnp.where(eq, jnp.float32(0.0), w1)
        a_v[pl.ds(k * L, L)] = lax.shift_right_logical(a, 2)
        b_v[pl.ds(k * L, L)] = lax.shift_right_logical(b, 2)
        am_v[pl.ds(k * L, L)] = (a & 3) * D
        bm_v[pl.ds(k * L, L)] = (b & 3) * D
        w0_v[pl.ds(k * L, L)] = w0
        w1_v[pl.ds(k * L, L)] = w1
    _scope.__exit__(None, None, None)

    with jax.named_scope("gather_rows"):
        c0 = pltpu.async_copy(odom_hbm.at[a_v], rows0_v, sem0)
        c1 = pltpu.async_copy(odom_hbm.at[b_v], rows1_v, sem1)
        c0.wait()
        c1.wait()

    with jax.named_scope("lerp"):
        lane = lax.iota(jnp.int32, L)
        for c in range(qpw * D // L):
            e = lane + (c * L)
            row = e // D
            col = e - row * D
            off0 = plsc.load_gather(am_v, [row])
            off1 = plsc.load_gather(bm_v, [row])
            y0 = plsc.load_gather(rows0_v, [row, off0 + col])
            y1 = plsc.load_gather(rows1_v, [row, off1 + col])
            s0 = plsc.load_gather(w0_v, [row])
            s1 = plsc.load_gather(w1_v, [row])
            plsc.store_scatter(out_v, [row, col], y0 * s0 + y1 * s1)

    with jax.named_scope("writeback"):
        pltpu.sync_copy(out_v, out_hbm.at[pl.ds(base, qpw)])


@jax.jit
def _run(vt, ot, odom):
    info = plsc.get_sparse_core_info()
    nc, ns = info.num_cores, info.num_subcores
    nw = nc * ns
    qpw = M // nw
    mesh = plsc.VectorSubcoreMesh(core_axis_name="c", subcore_axis_name="s")
    run = pl.kernel(
        functools.partial(_body, nc, qpw),
        out_type=jax.ShapeDtypeStruct((M, D), jnp.float32),
        mesh=mesh,
        compiler_params=pltpu.CompilerParams(
            needs_layout_passes=False, use_tc_tiling_on_sc=False),
        scratch_types=[
            pltpu.VMEM((N,), jnp.int32),
            pltpu.VMEM((qpw,), jnp.int32),
            pltpu.VMEM((qpw,), jnp.int32),
            pltpu.VMEM((qpw,), jnp.int32),
            pltpu.VMEM((qpw,), jnp.int32),
            pltpu.VMEM((qpw,), jnp.int32),
            pltpu.VMEM((qpw,), jnp.float32),
            pltpu.VMEM((qpw,), jnp.float32),
            pltpu.VMEM((qpw, 4 * D), jnp.float32),
            pltpu.VMEM((qpw, 4 * D), jnp.float32),
            pltpu.VMEM((qpw, D), jnp.float32),
            pltpu.SemaphoreType.DMA,
            pltpu.SemaphoreType.DMA,
        ],
    )
    return run(vt, ot, odom)


def kernel(valid_timestamps, odom_timestamps, odom):
    odom_q = odom.reshape(N // 4, 4 * D)
    return _run(valid_timestamps, odom_timestamps, odom_q)


# speedup vs baseline: 1.6082x; 1.6082x over previous
import functools

import jax
import jax.numpy as jnp
import numpy as np
from jax import lax
from jax.experimental import pallas as pl
from jax.experimental.pallas import tpu as pltpu
from jax.experimental.pallas import tpu_sc as plsc

M = 4096
N = 32768
L = 16
D = 12
IMAX = np.int32(2**31 - 1)


def _searchsorted(ot_v, target):
    pos = jnp.zeros((L,), jnp.int32)
    bit = N
    while bit >= 1:
        nxt = pos + bit
        ok = nxt <= N
        idx = jnp.minimum(nxt, N) - 1
        vals = plsc.load_gather(ot_v, [idx])
        pos = jnp.where(ok & (vals < target), nxt, pos)
        bit //= 2
    return pos


def _body(nc, qpw, vt_hbm, ot_hbm, odom_hbm, out_hbm,
          ot_v, vt_v, w0_v, w1_v, ei0_v, ei1_v, y0t_v, y1t_v,
          out_v, odom_sh, sem0, sem1):
    sid = lax.axis_index("s")
    wid = sid * nc + lax.axis_index("c")
    base = wid * qpw

    with jax.named_scope("stage_odom"):
        @pl.when(sid == 0)
        def _():
            pltpu.sync_copy(odom_hbm, odom_sh)

    with jax.named_scope("stage_table"):
        pltpu.sync_copy(ot_hbm, ot_v)
        pltpu.sync_copy(vt_hbm.at[pl.ds(base, qpw)], vt_v)

    _scope = jax.named_scope("search")
    _scope.__enter__()
    for k in range(qpw // L):
        vt16 = vt_v[pl.ds(k * L, L)]
        pL = _searchsorted(ot_v, vt16)
        wL = plsc.load_gather(ot_v, [jnp.maximum(pL, 1) - 1])
        wR = plsc.load_gather(ot_v, [jnp.minimum(pL, N - 1)])
        dL = vt16 - wL
        dR = jnp.where(pL < N, wR - vt16, IMAX)
        takeL = dL <= dR
        first_wL = _searchsorted(ot_v, wL)
        ref = jnp.where(takeL, first_wL, pL)
        d = jnp.where(takeL, dL, -dR)
        step = (d > 0).astype(jnp.int32) - (d < 0).astype(jnp.int32)
        q = jnp.clip(ref + step, 0, M - 1)
        a = jnp.minimum(ref, q)
        b = jnp.maximum(ref, q)
        x0 = plsc.load_gather(ot_v, [a])
        x1 = plsc.load_gather(ot_v, [b])
        eq = x0 == x1
        x0f = x0.astype(jnp.float32)
        x1f = x1.astype(jnp.float32)
        vtf = vt16.astype(jnp.float32)
        denom = jnp.where(eq, jnp.float32(1.0), x1f - x0f)
        w0 = 1.0 - (vtf - x0f) / denom
        w1 = 1.0 - w0
        w0 = jnp.where(eq, jnp.float32(1.0), w0)
        w1 = jnp.where(eq, jnp.float32(0.0), w1)
        w0_v[pl.ds(k * L, L)] = w0
        w1_v[pl.ds(k * L, L)] = w1
        aD = a * D
        bD = b * D
        for g in range(D):
            ei0_v[g, pl.ds(k * L, L)] = aD + g
            ei1_v[g, pl.ds(k * L, L)] = bD + g
    _scope.__exit__(None, None, None)

    with jax.named_scope("barrier"):
        plsc.subcore_barrier()

    with jax.named_scope("gather_rows"):
        handles = []
        for g in range(D):
            handles.append(
                pltpu.async_copy(odom_sh.at[ei0_v.at[g]], y0t_v.at[g], sem0))
            handles.append(
                pltpu.async_copy(odom_sh.at[ei1_v.at[g]], y1t_v.at[g], sem1))
        for h in handles:
            h.wait()

    with jax.named_scope("lerp"):
        lane = lax.iota(jnp.int32, L)
        for k in range(qpw // L):
            s0 = w0_v[pl.ds(k * L, L)]
            s1 = w1_v[pl.ds(k * L, L)]
            row = lane + (k * L)
            for g in range(D):
                y0 = y0t_v[g, pl.ds(k * L, L)]
                y1 = y1t_v[g, pl.ds(k * L, L)]
                col = jnp.full((L,), g, jnp.int32)
                plsc.store_scatter(out_v, [row, col], y0 * s0 + y1 * s1)

    with jax.named_scope("writeback"):
        pltpu.sync_copy(out_v, out_hbm.at[pl.ds(base, qpw)])


@jax.jit
def _run(vt, ot, odom_flat):
    info = plsc.get_sparse_core_info()
    nc, ns = info.num_cores, info.num_subcores
    nw = nc * ns
    qpw = M // nw
    mesh = plsc.VectorSubcoreMesh(core_axis_name="c", subcore_axis_name="s")
    run = pl.kernel(
        functools.partial(_body, nc, qpw),
        out_type=jax.ShapeDtypeStruct((M, D), jnp.float32),
        mesh=mesh,
        compiler_params=pltpu.CompilerParams(
            needs_layout_passes=False, use_tc_tiling_on_sc=False),
        scratch_types=[
            pltpu.VMEM((N,), jnp.int32),
            pltpu.VMEM((qpw,), jnp.int32),
            pltpu.VMEM((qpw,), jnp.float32),
            pltpu.VMEM((qpw,), jnp.float32),
            pltpu.VMEM((D, qpw), jnp.int32),
            pltpu.VMEM((D, qpw), jnp.int32),
            pltpu.VMEM((D, qpw), jnp.float32),
            pltpu.VMEM((D, qpw), jnp.float32),
            pltpu.VMEM((qpw, D), jnp.float32),
            pltpu.VMEM_SHARED((N * D,), jnp.float32),
            pltpu.SemaphoreType.DMA,
            pltpu.SemaphoreType.DMA,
        ],
    )
    return run(vt, ot, odom_flat)


def kernel(valid_timestamps, odom_timestamps, odom):
    return _run(valid_timestamps, odom_timestamps, odom.reshape(N * D))


# speedup vs baseline: 2.2305x; 1.3870x over previous
import functools

import jax
import jax.numpy as jnp
import numpy as np
from jax import lax
from jax.experimental import pallas as pl
from jax.experimental.pallas import tpu as pltpu
from jax.experimental.pallas import tpu_sc as plsc

M = 4096
N = 32768
L = 16
D = 12
IMAX = np.int32(2**31 - 1)


def _searchsorted(ot_v, target):
    pos = jnp.zeros((L,), jnp.int32)
    bit = N
    while bit >= 1:
        nxt = pos + bit
        ok = nxt <= N
        idx = jnp.minimum(nxt, N) - 1
        vals = plsc.load_gather(ot_v, [idx])
        pos = jnp.where(ok & (vals < target), nxt, pos)
        bit //= 2
    return pos


def _body(nc, qpw, vt_hbm, ot_hbm, odomt_hbm, out_hbm, *refs):
    cols_sh = refs[:D]
    (ot_v, vt_v, a_v, b_v, w0_v, w1_v, y0t_v, y1t_v, out_v,
     sem0, sem1) = refs[D:]
    sid = lax.axis_index("s")
    wid = sid * nc + lax.axis_index("c")
    base = wid * qpw

    with jax.named_scope("stage_odom"):
        for g in range(D):
            @pl.when(sid == g)
            def _(g=g):
                pltpu.sync_copy(odomt_hbm.at[g], cols_sh[g])

    with jax.named_scope("stage_table"):
        pltpu.sync_copy(ot_hbm, ot_v)
        pltpu.sync_copy(vt_hbm.at[pl.ds(base, qpw)], vt_v)

    _scope = jax.named_scope("search")
    _scope.__enter__()
    for k in range(qpw // L):
        vt16 = vt_v[pl.ds(k * L, L)]
        pL = _searchsorted(ot_v, vt16)
        wL = plsc.load_gather(ot_v, [jnp.maximum(pL, 1) - 1])
        wR = plsc.load_gather(ot_v, [jnp.minimum(pL, N - 1)])
        dL = vt16 - wL
        dR = jnp.where(pL < N, wR - vt16, IMAX)
        takeL = dL <= dR
        first_wL = _searchsorted(ot_v, wL)
        ref = jnp.where(takeL, first_wL, pL)
        d = jnp.where(takeL, dL, -dR)
        step = (d > 0).astype(jnp.int32) - (d < 0).astype(jnp.int32)
        q = jnp.clip(ref + step, 0, M - 1)
        a = jnp.minimum(ref, q)
        b = jnp.maximum(ref, q)
        x0 = plsc.load_gather(ot_v, [a])
        x1 = plsc.load_gather(ot_v, [b])
        eq = x0 == x1
        x0f = x0.astype(jnp.float32)
        x1f = x1.astype(jnp.float32)
        vtf = vt16.astype(jnp.float32)
        denom = jnp.where(eq, jnp.float32(1.0), x1f - x0f)
        w0 = 1.0 - (vtf - x0f) / denom
        w1 = 1.0 - w0
        w0 = jnp.where(eq, jnp.float32(1.0), w0)
        w1 = jnp.where(eq, jnp.float32(0.0), w1)
        a_v[pl.ds(k * L, L)] = a
        b_v[pl.ds(k * L, L)] = b
        w0_v[pl.ds(k * L, L)] = w0
        w1_v[pl.ds(k * L, L)] = w1
    _scope.__exit__(None, None, None)

    with jax.named_scope("barrier"):
        plsc.subcore_barrier()

    with jax.named_scope("gather_rows"):
        handles = []
        for g in range(D):
            handles.append(
                pltpu.async_copy(cols_sh[g].at[a_v], y0t_v.at[g], sem0))
            handles.append(
                pltpu.async_copy(cols_sh[g].at[b_v], y1t_v.at[g], sem1))
        for h in handles:
            h.wait()

    with jax.named_scope("lerp"):
        lane = lax.iota(jnp.int32, L)
        for k in range(qpw // L):
            s0 = w0_v[pl.ds(k * L, L)]
            s1 = w1_v[pl.ds(k * L, L)]
            row = lane + (k * L)
            for g in range(D):
                y0 = y0t_v[g, pl.ds(k * L, L)]
                y1 = y1t_v[g, pl.ds(k * L, L)]
                col = jnp.full((L,), g, jnp.int32)
                plsc.store_scatter(out_v, [row, col], y0 * s0 + y1 * s1)

    with jax.named_scope("writeback"):
        pltpu.sync_copy(out_v, out_hbm.at[pl.ds(base, qpw)])


@jax.jit
def _run(vt, ot, odomt):
    info = plsc.get_sparse_core_info()
    nc, ns = info.num_cores, info.num_subcores
    nw = nc * ns
    qpw = M // nw
    mesh = plsc.VectorSubcoreMesh(core_axis_name="c", subcore_axis_name="s")
    run = pl.kernel(
        functools.partial(_body, nc, qpw),
        out_type=jax.ShapeDtypeStruct((M, D), jnp.float32),
        mesh=mesh,
        compiler_params=pltpu.CompilerParams(
            needs_layout_passes=False, use_tc_tiling_on_sc=False),
        scratch_types=[pltpu.VMEM_SHARED((N,), jnp.float32)] * D + [
            pltpu.VMEM((N,), jnp.int32),
            pltpu.VMEM((qpw,), jnp.int32),
            pltpu.VMEM((qpw,), jnp.int32),
            pltpu.VMEM((qpw,), jnp.int32),
            pltpu.VMEM((qpw,), jnp.float32),
            pltpu.VMEM((qpw,), jnp.float32),
            pltpu.VMEM((D, qpw), jnp.float32),
            pltpu.VMEM((D, qpw), jnp.float32),
            pltpu.VMEM((qpw, D), jnp.float32),
            pltpu.SemaphoreType.DMA,
            pltpu.SemaphoreType.DMA,
        ],
    )
    return run(vt, ot, odomt)


def kernel(valid_timestamps, odom_timestamps, odom):
    return _run(valid_timestamps, odom_timestamps, odom.T)
